# baseline (speedup 1.0000x reference)
import functools

import jax
import jax.numpy as jnp
from jax import lax
from jax.experimental import pallas as pl
from jax.experimental.pallas import tpu as pltpu
from jax.experimental.pallas import tpu_sc as plsc

_W = 8
_BLK = 125
_NBUF = 4
_NC = 2
_NS = 16
_R = 1000
_RCH = 80


def _sc_mesh():
    return plsc.VectorSubcoreMesh(core_axis_name="c", subcore_axis_name="s")


def _rows_foreach_tile(s, n, fn):
    nch = n // _RCH
    trips = (nch + _NS - 1) // _NS

    def body(i, carry):
        cid = s + i * _NS

        @pl.when(cid < nch)
        def _():
            fn(pl.multiple_of(cid * _RCH, 8))

        return carry

    lax.fori_loop(0, trips, body, 0)


@functools.lru_cache(maxsize=None)
def _make_deg(n, e):
    nw = _NC * _NS
    ept = e // nw
    nblk = ept // _BLK

    @functools.partial(
        pl.kernel,
        mesh=_sc_mesh(),
        out_type=jax.ShapeDtypeStruct((_NC, n, _W), jnp.float32),
        compiler_params=pltpu.CompilerParams(use_tc_tiling_on_sc=False),
        scratch_types=[
            pltpu.VMEM((nblk, _BLK), jnp.int32),
            pltpu.VMEM((_BLK, _W), jnp.float32),
            pltpu.VMEM_SHARED((n, _W), jnp.float32),
            [pltpu.SemaphoreType.DMA] * 2,
        ],
    )
    def deg_kernel(dst_hbm, zeros_hbm, ones_hbm, out_hbm, didx2, ones_v, acc,
                   ssem):
        c = lax.axis_index("c")
        s = lax.axis_index("s")
        wid = c * _NS + s
        _rows_foreach_tile(s, n, lambda r0: pltpu.sync_copy(
            zeros_hbm.at[pl.ds(r0, _RCH)], acc.at[pl.ds(r0, _RCH)]))
        pltpu.sync_copy(ones_hbm, ones_v)
        pltpu.sync_copy(dst_hbm.at[wid], didx2)
        plsc.subcore_barrier()

        def slot(j, b):
            pltpu.async_copy(ones_v, acc.at[didx2.at[j]], ssem[b], add=True)

            @pl.when(j >= 2)
            def _():
                pltpu.make_async_copy(
                    ones_v, acc.at[didx2.at[j - 2]], ssem[b]).wait()

        def body(k, carry):
            slot(2 * k, 0)
            slot(2 * k + 1, 1)
            return carry

        lax.fori_loop(0, nblk // 2, body, 0)
        for b in range(2):
            pltpu.make_async_copy(
                ones_v, acc.at[didx2.at[nblk - 2 + b]], ssem[b]).wait()
        plsc.subcore_barrier()
        _rows_foreach_tile(s, n, lambda r0: pltpu.sync_copy(
            acc.at[pl.ds(r0, _RCH)], out_hbm.at[c, pl.ds(r0, _RCH)]))

    return deg_kernel


@functools.lru_cache(maxsize=None)
def _make_prop(n, e):
    nw = _NC * _NS
    ept = e // nw
    nblk = ept // _BLK

    @functools.partial(
        pl.kernel,
        mesh=_sc_mesh(),
        out_type=jax.ShapeDtypeStruct((_NC, n, _W), jnp.float32),
        compiler_params=pltpu.CompilerParams(use_tc_tiling_on_sc=False),
        scratch_types=[
            pltpu.VMEM((nblk, _BLK), jnp.int32),
            pltpu.VMEM((nblk, _BLK), jnp.int32),
            [pltpu.VMEM((_BLK, _W), jnp.float32)] * _NBUF,
            pltpu.VMEM_SHARED((n, _W), jnp.float32),
            [pltpu.SemaphoreType.DMA] * _NBUF,
            [pltpu.SemaphoreType.DMA] * _NBUF,
        ],
    )
    def prop_kernel(src_hbm, dst_hbm, g_hbm, zeros_hbm, out_hbm,
                    sidx2, didx2, msgs, acc, gsem, ssem):
        c = lax.axis_index("c")
        s = lax.axis_index("s")
        wid = c * _NS + s
        _rows_foreach_tile(s, n, lambda r0: pltpu.sync_copy(
            zeros_hbm.at[pl.ds(r0, _RCH)], acc.at[pl.ds(r0, _RCH)]))
        pltpu.sync_copy(src_hbm.at[wid], sidx2)
        pltpu.sync_copy(dst_hbm.at[wid], didx2)
        plsc.subcore_barrier()

        def gather_start(j, b):
            pltpu.async_copy(g_hbm.at[sidx2.at[j]], msgs[b], gsem[b])

        def gather_wait(j, b):
            pltpu.make_async_copy(g_hbm.at[sidx2.at[j]], msgs[b], gsem[b]).wait()

        def scatter_start(j, b):
            pltpu.async_copy(msgs[b], acc.at[didx2.at[j]], ssem[b], add=True)

        def scatter_wait(j, b):
            pltpu.make_async_copy(msgs[b], acc.at[didx2.at[j]], ssem[b]).wait()

        for b in range(_NBUF - 1):
            gather_start(b, b)

        def slot(j, b):
            gather_wait(j, b)
            scatter_start(j, b)
            pb = (b - 1) % _NBUF

            @pl.when(j >= 1)
            def _():
                scatter_wait(j - 1, pb)

            @pl.when(j + _NBUF - 1 < nblk)
            def _():
                gather_start(j + _NBUF - 1, pb)

        def body(k, carry):
            for b in range(_NBUF):
                slot(k * _NBUF + b, b)
            return carry

        lax.fori_loop(0, nblk // _NBUF, body, 0)
        scatter_wait(nblk - 1, (nblk - 1) % _NBUF)
        plsc.subcore_barrier()
        _rows_foreach_tile(s, n, lambda r0: pltpu.sync_copy(
            acc.at[pl.ds(r0, _RCH)], out_hbm.at[c, pl.ds(r0, _RCH)]))

    return prop_kernel


def _tc_stage1(x, w1, w2p, p0, p1):
    n, d = x.shape

    def k1(x_ref, w1_ref, w2p_ref, p0_ref, p1_ref, g0_ref, dinv_ref):
        w12 = jnp.dot(w1_ref[...], w2p_ref[...],
                      preferred_element_type=jnp.float32)
        z16 = jnp.dot(x_ref[...], w12, preferred_element_type=jnp.float32)
        deg = p0_ref[...] + p1_ref[...] + 1.0
        dinv = lax.rsqrt(deg)
        col = lax.broadcasted_iota(jnp.int32, z16.shape, 1)
        zt = z16 + jnp.where(col == 2, 1.0, 0.0)
        g0_ref[...] = dinv * zt
        dinv_ref[...] = dinv

    return pl.pallas_call(
        k1,
        grid=(n // _R,),
        in_specs=[
            pl.BlockSpec((_R, d), lambda i: (i, 0)),
            pl.BlockSpec((d, d), lambda i: (0, 0)),
            pl.BlockSpec((d, _W), lambda i: (0, 0)),
            pl.BlockSpec((_R, _W), lambda i: (i, 0)),
            pl.BlockSpec((_R, _W), lambda i: (i, 0)),
        ],
        out_specs=[
            pl.BlockSpec((_R, _W), lambda i: (i, 0)),
            pl.BlockSpec((_R, _W), lambda i: (i, 0)),
        ],
        out_shape=[
            jax.ShapeDtypeStruct((n, _W), jnp.float32),
            jax.ShapeDtypeStruct((n, _W), jnp.float32),
        ],
    )(x, w1, w2p, p0, p1)


def _tc_stage2(q0, q1, g0, dinv, b1r, w2p, b2p):
    n = q0.shape[0]
    d = b1r.shape[1]

    def k2(q0_ref, q1_ref, g0_ref, dinv_ref, b1_ref, w2p_ref, b2_ref,
           g1_ref, st_ref):
        t1 = q0_ref[...] + q1_ref[...] + g0_ref[...]
        dv = dinv_ref[...]
        g1_ref[...] = dv * dv * t1
        c16 = jnp.dot(b1_ref[...], w2p_ref[...],
                      preferred_element_type=jnp.float32)
        st_ref[...] = dv * t1[:, 2:3] * c16 + b2_ref[...]

    return pl.pallas_call(
        k2,
        grid=(n // _R,),
        in_specs=[
            pl.BlockSpec((_R, _W), lambda i: (i, 0)),
            pl.BlockSpec((_R, _W), lambda i: (i, 0)),
            pl.BlockSpec((_R, _W), lambda i: (i, 0)),
            pl.BlockSpec((_R, _W), lambda i: (i, 0)),
            pl.BlockSpec((1, d), lambda i: (0, 0)),
            pl.BlockSpec((d, _W), lambda i: (0, 0)),
            pl.BlockSpec((1, _W), lambda i: (0, 0)),
        ],
        out_specs=[
            pl.BlockSpec((_R, _W), lambda i: (i, 0)),
            pl.BlockSpec((_R, _W), lambda i: (i, 0)),
        ],
        out_shape=[
            jax.ShapeDtypeStruct((n, _W), jnp.float32),
            jax.ShapeDtypeStruct((n, _W), jnp.float32),
        ],
    )(q0, q1, g0, dinv, b1r, w2p, b2p)


def _tc_stage3(r0, r1, g1, dinv, st):
    n = r0.shape[0]

    def k3(r0_ref, r1_ref, g1_ref, dinv_ref, st_ref, out_ref):
        t2 = r0_ref[...] + r1_ref[...] + g1_ref[...]
        y = dinv_ref[...] * t2 + st_ref[...]
        a = y[:, 0:1]
        b = y[:, 1:2]
        m = jnp.maximum(a, b)
        ea = jnp.exp(a - m)
        eb = jnp.exp(b - m)
        tot = ea + eb
        col = lax.broadcasted_iota(jnp.int32, (_R, 2), 1)
        out_ref[...] = jnp.where(col == 0, ea / tot, eb / tot)

    return pl.pallas_call(
        k3,
        grid=(n // _R,),
        in_specs=[
            pl.BlockSpec((_R, _W), lambda i: (i, 0)),
            pl.BlockSpec((_R, _W), lambda i: (i, 0)),
            pl.BlockSpec((_R, _W), lambda i: (i, 0)),
            pl.BlockSpec((_R, _W), lambda i: (i, 0)),
            pl.BlockSpec((_R, _W), lambda i: (i, 0)),
        ],
        out_specs=pl.BlockSpec((_R, 2), lambda i: (i, 0)),
        out_shape=jax.ShapeDtypeStruct((n, 2), jnp.float32),
    )(r0, r1, g1, dinv, st)


def kernel(X, edge_index, W1, b1, W2, b2):
    n, _ = X.shape
    e = edge_index.shape[1]
    nw = _NC * _NS
    nblk = e // nw // _BLK
    src = edge_index[0].reshape(nw, nblk, _BLK)
    dst = edge_index[1].reshape(nw, nblk, _BLK)

    w2p = jnp.pad(W2, ((0, 0), (0, _W - W2.shape[1])))
    b1r = b1.reshape(1, -1)
    b2p = jnp.pad(b2.reshape(1, -1), ((0, 0), (0, _W - b2.shape[0])))
    zeros_t = jnp.zeros((n, _W), jnp.float32)
    ones_t = jnp.ones((_BLK, _W), jnp.float32)

    degp = _make_deg(n, e)(dst, zeros_t, ones_t)
    g0, dinv = _tc_stage1(X, W1, w2p, degp[0], degp[1])
    t1p = _make_prop(n, e)(src, dst, g0, zeros_t)
    g1, st = _tc_stage2(t1p[0], t1p[1], g0, dinv, b1r, w2p, b2p)
    t2p = _make_prop(n, e)(src, dst, g1, zeros_t)
    return _tc_stage3(t2p[0], t2p[1], g1, dinv, st)


# speedup vs baseline: 1.0455x; 1.0455x over previous
import functools

import jax
import jax.numpy as jnp
from jax import lax
from jax.experimental import pallas as pl
from jax.experimental.pallas import tpu as pltpu
from jax.experimental.pallas import tpu_sc as plsc

_W = 8
_BLK = 125
_NBUF = 8
_LEAD = 4
_NC = 2
_NS = 16
_R = 1000
_RCH = 80


def _sc_mesh():
    return plsc.VectorSubcoreMesh(core_axis_name="c", subcore_axis_name="s")


def _rows_foreach_tile(s, n, fn):
    nch = n // _RCH
    trips = (nch + _NS - 1) // _NS

    def body(i, carry):
        cid = s + i * _NS

        @pl.when(cid < nch)
        def _():
            fn(pl.multiple_of(cid * _RCH, 8))

        return carry

    lax.fori_loop(0, trips, body, 0)


@functools.lru_cache(maxsize=None)
def _make_deg(n, e):
    nw = _NC * _NS
    ept = e // nw
    nblk = ept // _BLK

    @functools.partial(
        pl.kernel,
        mesh=_sc_mesh(),
        out_type=jax.ShapeDtypeStruct((_NC, n, _W), jnp.float32),
        compiler_params=pltpu.CompilerParams(use_tc_tiling_on_sc=False),
        scratch_types=[
            pltpu.VMEM((nblk, _BLK), jnp.int32),
            pltpu.VMEM((_BLK, _W), jnp.float32),
            pltpu.VMEM_SHARED((n, _W), jnp.float32),
            [pltpu.SemaphoreType.DMA] * 2,
        ],
    )
    def deg_kernel(dst_hbm, zeros_hbm, ones_hbm, out_hbm, didx2, ones_v, acc,
                   ssem):
        c = lax.axis_index("c")
        s = lax.axis_index("s")
        wid = c * _NS + s
        _rows_foreach_tile(s, n, lambda r0: pltpu.sync_copy(
            zeros_hbm.at[pl.ds(r0, _RCH)], acc.at[pl.ds(r0, _RCH)]))
        pltpu.sync_copy(ones_hbm, ones_v)
        pltpu.sync_copy(dst_hbm.at[wid], didx2)
        plsc.subcore_barrier()

        def slot(j, b):
            pltpu.async_copy(ones_v, acc.at[didx2.at[j]], ssem[b], add=True)

            @pl.when(j >= 2)
            def _():
                pltpu.make_async_copy(
                    ones_v, acc.at[didx2.at[j - 2]], ssem[b]).wait()

        def body(k, carry):
            slot(2 * k, 0)
            slot(2 * k + 1, 1)
            return carry

        lax.fori_loop(0, nblk // 2, body, 0)
        for b in range(2):
            pltpu.make_async_copy(
                ones_v, acc.at[didx2.at[nblk - 2 + b]], ssem[b]).wait()
        plsc.subcore_barrier()
        _rows_foreach_tile(s, n, lambda r0: pltpu.sync_copy(
            acc.at[pl.ds(r0, _RCH)], out_hbm.at[c, pl.ds(r0, _RCH)]))

    return deg_kernel


@functools.lru_cache(maxsize=None)
def _make_prop(n, e):
    nw = _NC * _NS
    ept = e // nw
    nblk = ept // _BLK

    @functools.partial(
        pl.kernel,
        mesh=_sc_mesh(),
        out_type=jax.ShapeDtypeStruct((_NC, n, _W), jnp.float32),
        compiler_params=pltpu.CompilerParams(use_tc_tiling_on_sc=False),
        scratch_types=[
            pltpu.VMEM((nblk, _BLK), jnp.int32),
            pltpu.VMEM((nblk, _BLK), jnp.int32),
            [pltpu.VMEM((_BLK, _W), jnp.float32)] * _NBUF,
            pltpu.VMEM_SHARED((n, _W), jnp.float32),
            [pltpu.SemaphoreType.DMA] * _NBUF,
            [pltpu.SemaphoreType.DMA] * _NBUF,
        ],
    )
    def prop_kernel(src_hbm, dst_hbm, g_hbm, zeros_hbm, out_hbm,
                    sidx2, didx2, msgs, acc, gsem, ssem):
        c = lax.axis_index("c")
        s = lax.axis_index("s")
        wid = c * _NS + s
        _rows_foreach_tile(s, n, lambda r0: pltpu.sync_copy(
            zeros_hbm.at[pl.ds(r0, _RCH)], acc.at[pl.ds(r0, _RCH)]))
        pltpu.sync_copy(src_hbm.at[wid], sidx2)
        pltpu.sync_copy(dst_hbm.at[wid], didx2)
        plsc.subcore_barrier()

        def gather_start(j, b):
            pltpu.async_copy(g_hbm.at[sidx2.at[j]], msgs[b], gsem[b])

        def gather_wait(j, b):
            pltpu.make_async_copy(g_hbm.at[sidx2.at[j]], msgs[b], gsem[b]).wait()

        def scatter_start(j, b):
            pltpu.async_copy(msgs[b], acc.at[didx2.at[j]], ssem[b], add=True)

        def scatter_wait(j, b):
            pltpu.make_async_copy(msgs[b], acc.at[didx2.at[j]], ssem[b]).wait()

        for b in range(_LEAD):
            gather_start(b, b)

        def slot(j, b):
            gather_wait(j, b)
            scatter_start(j, b)
            pb = (b - _LEAD) % _NBUF

            @pl.when(j >= _LEAD)
            def _():
                scatter_wait(j - _LEAD, pb)

            @pl.when(j + _LEAD < nblk)
            def _():
                gather_start(j + _LEAD, pb)

        def body(k, carry):
            for b in range(_NBUF):
                slot(k * _NBUF + b, b)
            return carry

        lax.fori_loop(0, nblk // _NBUF, body, 0)
        for j in range(nblk - _LEAD, nblk):
            scatter_wait(j, j % _NBUF)
        plsc.subcore_barrier()
        _rows_foreach_tile(s, n, lambda r0: pltpu.sync_copy(
            acc.at[pl.ds(r0, _RCH)], out_hbm.at[c, pl.ds(r0, _RCH)]))

    return prop_kernel


def _tc_stage1(x, w1, w2p, p0, p1):
    n, d = x.shape

    def k1(x_ref, w1_ref, w2p_ref, p0_ref, p1_ref, g0_ref, dinv_ref):
        w12 = jnp.dot(w1_ref[...], w2p_ref[...],
                      preferred_element_type=jnp.float32)
        z16 = jnp.dot(x_ref[...], w12, preferred_element_type=jnp.float32)
        deg = p0_ref[...] + p1_ref[...] + 1.0
        dinv = lax.rsqrt(deg)
        col = lax.broadcasted_iota(jnp.int32, z16.shape, 1)
        zt = z16 + jnp.where(col == 2, 1.0, 0.0)
        g0_ref[...] = dinv * zt
        dinv_ref[...] = dinv

    return pl.pallas_call(
        k1,
        grid=(n // _R,),
        in_specs=[
            pl.BlockSpec((_R, d), lambda i: (i, 0)),
            pl.BlockSpec((d, d), lambda i: (0, 0)),
            pl.BlockSpec((d, _W), lambda i: (0, 0)),
            pl.BlockSpec((_R, _W), lambda i: (i, 0)),
            pl.BlockSpec((_R, _W), lambda i: (i, 0)),
        ],
        out_specs=[
            pl.BlockSpec((_R, _W), lambda i: (i, 0)),
            pl.BlockSpec((_R, _W), lambda i: (i, 0)),
        ],
        out_shape=[
            jax.ShapeDtypeStruct((n, _W), jnp.float32),
            jax.ShapeDtypeStruct((n, _W), jnp.float32),
        ],
    )(x, w1, w2p, p0, p1)


def _tc_stage2(q0, q1, g0, dinv, b1r, w2p, b2p):
    n = q0.shape[0]
    d = b1r.shape[1]

    def k2(q0_ref, q1_ref, g0_ref, dinv_ref, b1_ref, w2p_ref, b2_ref,
           g1_ref, st_ref):
        t1 = q0_ref[...] + q1_ref[...] + g0_ref[...]
        dv = dinv_ref[...]
        g1_ref[...] = dv * dv * t1
        c16 = jnp.dot(b1_ref[...], w2p_ref[...],
                      preferred_element_type=jnp.float32)
        st_ref[...] = dv * t1[:, 2:3] * c16 + b2_ref[...]

    return pl.pallas_call(
        k2,
        grid=(n // _R,),
        in_specs=[
            pl.BlockSpec((_R, _W), lambda i: (i, 0)),
            pl.BlockSpec((_R, _W), lambda i: (i, 0)),
            pl.BlockSpec((_R, _W), lambda i: (i, 0)),
            pl.BlockSpec((_R, _W), lambda i: (i, 0)),
            pl.BlockSpec((1, d), lambda i: (0, 0)),
            pl.BlockSpec((d, _W), lambda i: (0, 0)),
            pl.BlockSpec((1, _W), lambda i: (0, 0)),
        ],
        out_specs=[
            pl.BlockSpec((_R, _W), lambda i: (i, 0)),
            pl.BlockSpec((_R, _W), lambda i: (i, 0)),
        ],
        out_shape=[
            jax.ShapeDtypeStruct((n, _W), jnp.float32),
            jax.ShapeDtypeStruct((n, _W), jnp.float32),
        ],
    )(q0, q1, g0, dinv, b1r, w2p, b2p)


def _tc_stage3(r0, r1, g1, dinv, st):
    n = r0.shape[0]

    def k3(r0_ref, r1_ref, g1_ref, dinv_ref, st_ref, out_ref):
        t2 = r0_ref[...] + r1_ref[...] + g1_ref[...]
        y = dinv_ref[...] * t2 + st_ref[...]
        a = y[:, 0:1]
        b = y[:, 1:2]
        m = jnp.maximum(a, b)
        ea = jnp.exp(a - m)
        eb = jnp.exp(b - m)
        tot = ea + eb
        col = lax.broadcasted_iota(jnp.int32, (_R, 2), 1)
        out_ref[...] = jnp.where(col == 0, ea / tot, eb / tot)

    return pl.pallas_call(
        k3,
        grid=(n // _R,),
        in_specs=[
            pl.BlockSpec((_R, _W), lambda i: (i, 0)),
            pl.BlockSpec((_R, _W), lambda i: (i, 0)),
            pl.BlockSpec((_R, _W), lambda i: (i, 0)),
            pl.BlockSpec((_R, _W), lambda i: (i, 0)),
            pl.BlockSpec((_R, _W), lambda i: (i, 0)),
        ],
        out_specs=pl.BlockSpec((_R, 2), lambda i: (i, 0)),
        out_shape=jax.ShapeDtypeStruct((n, 2), jnp.float32),
    )(r0, r1, g1, dinv, st)


def kernel(X, edge_index, W1, b1, W2, b2):
    n, _ = X.shape
    e = edge_index.shape[1]
    nw = _NC * _NS
    nblk = e // nw // _BLK
    src = edge_index[0].reshape(nw, nblk, _BLK)
    dst = edge_index[1].reshape(nw, nblk, _BLK)

    w2p = jnp.pad(W2, ((0, 0), (0, _W - W2.shape[1])))
    b1r = b1.reshape(1, -1)
    b2p = jnp.pad(b2.reshape(1, -1), ((0, 0), (0, _W - b2.shape[0])))
    zeros_t = jnp.zeros((n, _W), jnp.float32)
    ones_t = jnp.ones((_BLK, _W), jnp.float32)

    degp = _make_deg(n, e)(dst, zeros_t, ones_t)
    g0, dinv = _tc_stage1(X, W1, w2p, degp[0], degp[1])
    t1p = _make_prop(n, e)(src, dst, g0, zeros_t)
    g1, st = _tc_stage2(t1p[0], t1p[1], g0, dinv, b1r, w2p, b2p)
    t2p = _make_prop(n, e)(src, dst, g1, zeros_t)
    return _tc_stage3(t2p[0], t2p[1], g1, dinv, st)


# speedup vs baseline: 1.1346x; 1.0852x over previous
import functools

import jax
import jax.numpy as jnp
from jax import lax
from jax.experimental import pallas as pl
from jax.experimental.pallas import tpu as pltpu
from jax.experimental.pallas import tpu_sc as plsc

_W = 8
_BLK = 125
_NBUF = 8
_LEAD = 4
_NC = 2
_NS = 16
_R = 1000
_RCH = 80

_MAGIC = 0x5F3759DF


def _sc_mesh():
    return plsc.VectorSubcoreMesh(core_axis_name="c", subcore_axis_name="s")


def _sc_params():
    return pltpu.CompilerParams(
        use_tc_tiling_on_sc=False, needs_layout_passes=False)


def _chunks_foreach_tile(s, n, fn):
    nch = n // _RCH
    trips = (nch + _NS - 1) // _NS

    def body(i, carry):
        cid = s + i * _NS

        @pl.when(cid < nch)
        def _():
            fn(pl.multiple_of(cid * _RCH, 8))

        return carry

    lax.fori_loop(0, trips, body, 0)


def _flat_idx(v):
    io = lax.iota(jnp.int32, 16)
    return 2 * v + (io >> 3), io & 7


def _rsqrt_sc(d):
    i = plsc.bitcast(d, jnp.int32)
    y = plsc.bitcast(_MAGIC - (i >> 1), jnp.float32)
    for _ in range(3):
        y = y * (1.5 - 0.5 * d * y * y)
    return y


def _prop_ring(nblk, gtab, sidx2, didx2, acc, msgs, gsem, ssem):
    def gather_start(j, b):
        pltpu.async_copy(gtab.at[sidx2.at[j]], msgs[b], gsem[b])

    def gather_wait(j, b):
        pltpu.make_async_copy(gtab.at[sidx2.at[j]], msgs[b], gsem[b]).wait()

    def scatter_start(j, b):
        pltpu.async_copy(msgs[b], acc.at[didx2.at[j]], ssem[b], add=True)

    def scatter_wait(j, b):
        pltpu.make_async_copy(msgs[b], acc.at[didx2.at[j]], ssem[b]).wait()

    for b in range(_LEAD):
        gather_start(b, b)

    def slot(j, b):
        gather_wait(j, b)
        scatter_start(j, b)
        pb = (b - _LEAD) % _NBUF

        @pl.when(j >= _LEAD)
        def _():
            scatter_wait(j - _LEAD, pb)

        @pl.when(j + _LEAD < nblk)
        def _():
            gather_start(j + _LEAD, pb)

    def body(k, carry):
        for b in range(_NBUF):
            slot(k * _NBUF + b, b)
        return carry

    lax.fori_loop(0, nblk // _NBUF, body, 0)
    for j in range(nblk - _LEAD, nblk):
        scatter_wait(j, j % _NBUF)


@functools.lru_cache(maxsize=None)
def _make_sc1(n, e):
    nw = _NC * _NS
    nblk = e // nw // _BLK
    dnblk = e // _NS // _BLK

    @functools.partial(
        pl.kernel,
        mesh=_sc_mesh(),
        out_type=[
            jax.ShapeDtypeStruct((_NC, n, _W), jnp.float32),
            jax.ShapeDtypeStruct((n, _W), jnp.float32),
        ],
        compiler_params=_sc_params(),
        scratch_types=[
            pltpu.VMEM((dnblk, _BLK), jnp.int32),
            pltpu.VMEM((nblk, _BLK), jnp.int32),
            pltpu.VMEM((nblk, _BLK), jnp.int32),
            pltpu.VMEM((_BLK, _W), jnp.float32),
            [pltpu.VMEM((_BLK, _W), jnp.float32)] * _NBUF,
            pltpu.VMEM((_RCH, _W), jnp.float32),
            pltpu.VMEM((_RCH, _W), jnp.float32),
            pltpu.VMEM((_RCH, _W), jnp.float32),
            pltpu.VMEM((_RCH, _W), jnp.float32),
            pltpu.VMEM_SHARED((n, _W), jnp.float32),
            pltpu.VMEM_SHARED((n, _W), jnp.float32),
            pltpu.VMEM_SHARED((n, _W), jnp.float32),
            [pltpu.SemaphoreType.DMA] * 2,
            [pltpu.SemaphoreType.DMA] * _NBUF,
            [pltpu.SemaphoreType.DMA] * _NBUF,
        ],
    )
    def sc1(dstf_hbm, src_hbm, dst_hbm, zt_hbm, zeros_hbm, ones_hbm,
            t1p_hbm, dinv_hbm,
            didxf, sidx2, didx2, ones_v, msgs, degc, ztc, g0c, dvc,
            accd, g0tab, acc1, dsem, gsem, ssem):
        c = lax.axis_index("c")
        s = lax.axis_index("s")
        wid = c * _NS + s
        pltpu.sync_copy(dstf_hbm.at[s], didxf)
        pltpu.sync_copy(src_hbm.at[wid], sidx2)
        pltpu.sync_copy(dst_hbm.at[wid], didx2)
        pltpu.sync_copy(ones_hbm, ones_v)
        _chunks_foreach_tile(s, n, lambda r0: pltpu.sync_copy(
            zeros_hbm.at[pl.ds(r0, _RCH)], accd.at[pl.ds(r0, _RCH)]))

        def zero_acc1(r0):
            @pl.when(c == 1)
            def _():
                pltpu.sync_copy(
                    zeros_hbm.at[pl.ds(r0, _RCH)], acc1.at[pl.ds(r0, _RCH)])

        _chunks_foreach_tile(s, n, zero_acc1)
        plsc.subcore_barrier()

        def dslot(j, b):
            pltpu.async_copy(ones_v, accd.at[didxf.at[j]], dsem[b], add=True)

            @pl.when(j >= 2)
            def _():
                pltpu.make_async_copy(
                    ones_v, accd.at[didxf.at[j - 2]], dsem[b]).wait()

        def dbody(k, carry):
            dslot(2 * k, 0)
            dslot(2 * k + 1, 1)
            return carry

        lax.fori_loop(0, dnblk // 2, dbody, 0)
        for b in range(2):
            pltpu.make_async_copy(
                ones_v, accd.at[didxf.at[dnblk - 2 + b]], dsem[b]).wait()
        plsc.subcore_barrier()

        def build(r0):
            pltpu.sync_copy(accd.at[pl.ds(r0, _RCH)], degc)
            pltpu.sync_copy(zt_hbm.at[pl.ds(r0, _RCH)], ztc)

            def vbody(v, carry):
                row, col = _flat_idx(v)
                d = plsc.load_gather(degc, [row, col]) + 1.0
                y = _rsqrt_sc(d)
                z = plsc.load_gather(ztc, [row, col])
                plsc.store_scatter(g0c, [row, col], y * z)
                plsc.store_scatter(dvc, [row, col], y)
                return carry

            lax.fori_loop(0, _RCH * _W // 16, vbody, 0)
            pltpu.sync_copy(g0c, g0tab.at[pl.ds(r0, _RCH)])

            @pl.when(c == 0)
            def _():
                pltpu.sync_copy(g0c, acc1.at[pl.ds(r0, _RCH)])
                pltpu.sync_copy(dvc, dinv_hbm.at[pl.ds(r0, _RCH)])

        _chunks_foreach_tile(s, n, build)
        plsc.subcore_barrier()

        _prop_ring(nblk, g0tab, sidx2, didx2, acc1, msgs, gsem, ssem)
        plsc.subcore_barrier()

        _chunks_foreach_tile(s, n, lambda r0: pltpu.sync_copy(
            acc1.at[pl.ds(r0, _RCH)], t1p_hbm.at[c, pl.ds(r0, _RCH)]))

    return sc1


@functools.lru_cache(maxsize=None)
def _make_sc2(n, e):
    nw = _NC * _NS
    nblk = e // nw // _BLK

    @functools.partial(
        pl.kernel,
        mesh=_sc_mesh(),
        out_type=[
            jax.ShapeDtypeStruct((_NC, n, _W), jnp.float32),
            jax.ShapeDtypeStruct((n, _W), jnp.float32),
        ],
        compiler_params=_sc_params(),
        scratch_types=[
            pltpu.VMEM((nblk, _BLK), jnp.int32),
            pltpu.VMEM((nblk, _BLK), jnp.int32),
            [pltpu.VMEM((_BLK, _W), jnp.float32)] * _NBUF,
            pltpu.VMEM((_RCH, _W), jnp.float32),
            pltpu.VMEM((_RCH, _W), jnp.float32),
            pltpu.VMEM((_RCH, _W), jnp.float32),
            pltpu.VMEM((_RCH, _W), jnp.float32),
            pltpu.VMEM((_RCH, _W), jnp.float32),
            pltpu.VMEM_SHARED((n, _W), jnp.float32),
            pltpu.VMEM_SHARED((n, _W), jnp.float32),
            [pltpu.SemaphoreType.DMA] * _NBUF,
            [pltpu.SemaphoreType.DMA] * _NBUF,
        ],
    )
    def sc2(src_hbm, dst_hbm, q0_hbm, q1_hbm, dinv_hbm, zeros_hbm,
            rp_hbm, taux_hbm,
            sidx2, didx2, msgs, q0c, q1c, dvc, g1c, txc,
            g1tab, acc2, gsem, ssem):
        c = lax.axis_index("c")
        s = lax.axis_index("s")
        wid = c * _NS + s
        pltpu.sync_copy(src_hbm.at[wid], sidx2)
        pltpu.sync_copy(dst_hbm.at[wid], didx2)

        def build(r0):
            pltpu.sync_copy(q0_hbm.at[pl.ds(r0, _RCH)], q0c)
            pltpu.sync_copy(q1_hbm.at[pl.ds(r0, _RCH)], q1c)
            pltpu.sync_copy(dinv_hbm.at[pl.ds(r0, _RCH)], dvc)

            def vbody(v, carry):
                row, col = _flat_idx(v)
                t1 = (plsc.load_gather(q0c, [row, col])
                      + plsc.load_gather(q1c, [row, col]))
                dv = plsc.load_gather(dvc, [row, col])
                plsc.store_scatter(g1c, [row, col], dv * dv * t1)
                plsc.store_scatter(txc, [row, col], dv * t1)
                return carry

            lax.fori_loop(0, _RCH * _W // 16, vbody, 0)
            pltpu.sync_copy(g1c, g1tab.at[pl.ds(r0, _RCH)])

            @pl.when(c == 0)
            def _():
                pltpu.sync_copy(g1c, acc2.at[pl.ds(r0, _RCH)])
                pltpu.sync_copy(txc, taux_hbm.at[pl.ds(r0, _RCH)])

            @pl.when(c == 1)
            def _():
                pltpu.sync_copy(
                    zeros_hbm.at[pl.ds(r0, _RCH)], acc2.at[pl.ds(r0, _RCH)])

        _chunks_foreach_tile(s, n, build)
        plsc.subcore_barrier()

        _prop_ring(nblk, g1tab, sidx2, didx2, acc2, msgs, gsem, ssem)
        plsc.subcore_barrier()

        _chunks_foreach_tile(s, n, lambda r0: pltpu.sync_copy(
            acc2.at[pl.ds(r0, _RCH)], rp_hbm.at[c, pl.ds(r0, _RCH)]))

    return sc2


def _tc_stage_a(x, w1, w2p):
    n, d = x.shape

    def ka(x_ref, w1_ref, w2p_ref, zt_ref):
        w12 = jnp.dot(w1_ref[...], w2p_ref[...],
                      preferred_element_type=jnp.float32)
        z16 = jnp.dot(x_ref[...], w12, preferred_element_type=jnp.float32)
        col = lax.broadcasted_iota(jnp.int32, z16.shape, 1)
        zt_ref[...] = z16 + jnp.where(col == 2, 1.0, 0.0)

    return pl.pallas_call(
        ka,
        grid=(n // _R,),
        in_specs=[
            pl.BlockSpec((_R, d), lambda i: (i, 0)),
            pl.BlockSpec((d, d), lambda i: (0, 0)),
            pl.BlockSpec((d, _W), lambda i: (0, 0)),
        ],
        out_specs=pl.BlockSpec((_R, _W), lambda i: (i, 0)),
        out_shape=jax.ShapeDtypeStruct((n, _W), jnp.float32),
    )(x, w1, w2p)


def _tc_stage_b(r0, r1, dinv, taux, b1r, w2p, b2p):
    n = r0.shape[0]
    d = b1r.shape[1]

    def kb(r0_ref, r1_ref, dinv_ref, taux_ref, b1_ref, w2p_ref, b2_ref,
           out_ref):
        c16 = jnp.dot(b1_ref[...], w2p_ref[...],
                      preferred_element_type=jnp.float32)
        t2 = r0_ref[...] + r1_ref[...]
        y = dinv_ref[...] * t2 + taux_ref[:, 2:3] * c16 + b2_ref[...]
        a = y[:, 0:1]
        b = y[:, 1:2]
        m = jnp.maximum(a, b)
        ea = jnp.exp(a - m)
        eb = jnp.exp(b - m)
        tot = ea + eb
        col = lax.broadcasted_iota(jnp.int32, (_R, 2), 1)
        out_ref[...] = jnp.where(col == 0, ea / tot, eb / tot)

    return pl.pallas_call(
        kb,
        grid=(n // _R,),
        in_specs=[
            pl.BlockSpec((_R, _W), lambda i: (i, 0)),
            pl.BlockSpec((_R, _W), lambda i: (i, 0)),
            pl.BlockSpec((_R, _W), lambda i: (i, 0)),
            pl.BlockSpec((_R, _W), lambda i: (i, 0)),
            pl.BlockSpec((1, d), lambda i: (0, 0)),
            pl.BlockSpec((d, _W), lambda i: (0, 0)),
            pl.BlockSpec((1, _W), lambda i: (0, 0)),
        ],
        out_specs=pl.BlockSpec((_R, 2), lambda i: (i, 0)),
        out_shape=jax.ShapeDtypeStruct((n, 2), jnp.float32),
    )(r0, r1, dinv, taux, b1r, w2p, b2p)


def kernel(X, edge_index, W1, b1, W2, b2):
    n, _ = X.shape
    e = edge_index.shape[1]
    nw = _NC * _NS
    nblk = e // nw // _BLK
    dnblk = e // _NS // _BLK
    src = edge_index[0].reshape(nw, nblk, _BLK)
    dst = edge_index[1].reshape(nw, nblk, _BLK)
    dstf = edge_index[1].reshape(_NS, dnblk, _BLK)

    w2p = jnp.pad(W2, ((0, 0), (0, _W - W2.shape[1])))
    b1r = b1.reshape(1, -1)
    b2p = jnp.pad(b2.reshape(1, -1), ((0, 0), (0, _W - b2.shape[0])))
    zeros_t = jnp.zeros((n, _W), jnp.float32)
    ones_t = jnp.ones((_BLK, _W), jnp.float32)

    zt = _tc_stage_a(X, W1, w2p)
    t1p, dinv = _make_sc1(n, e)(dstf, src, dst, zt, zeros_t, ones_t)
    rp, taux = _make_sc2(n, e)(src, dst, t1p[0], t1p[1], dinv, zeros_t)
    return _tc_stage_b(rp[0], rp[1], dinv, taux, b1r, w2p, b2p)


# speedup vs baseline: 1.9098x; 1.6833x over previous
import functools

import jax
import jax.numpy as jnp
from jax import lax
from jax.experimental import pallas as pl
from jax.experimental.pallas import tpu as pltpu
from jax.experimental.pallas import tpu_sc as plsc

_W = 8
_BLK = 125
_NBUF = 8
_LEAD = 4
_NC = 2
_NS = 16
_R = 1000
_RCH = 80

_MAGIC = 0x5F3759DF


def _sc_mesh():
    return plsc.VectorSubcoreMesh(core_axis_name="c", subcore_axis_name="s")


def _sc_params():
    return pltpu.CompilerParams(
        use_tc_tiling_on_sc=False, needs_layout_passes=False)


def _chunks_foreach_tile(s, n, fn):
    nch = n // _RCH
    trips = (nch + _NS - 1) // _NS

    def body(i, carry):
        cid = s + i * _NS

        @pl.when(cid < nch)
        def _():
            fn(cid, pl.multiple_of(cid * _RCH, 8))

        return carry

    lax.fori_loop(0, trips, body, 0)


def _flat_idx(v):
    io = lax.iota(jnp.int32, 16)
    return 2 * v + (io >> 3), io & 7


def _rsqrt_sc(d):
    i = plsc.bitcast(d, jnp.int32)
    y = plsc.bitcast(_MAGIC - (i >> 1), jnp.float32)
    for _ in range(3):
        y = y * (1.5 - 0.5 * d * y * y)
    return y


def _prop_ring(nblk, gtab, sidx2, didx2, acc, msgs, gsem, ssem):
    def gather_start(j, b):
        pltpu.async_copy(gtab.at[sidx2.at[j]], msgs[b], gsem[b])

    def gather_wait(j, b):
        pltpu.make_async_copy(gtab.at[sidx2.at[j]], msgs[b], gsem[b]).wait()

    def scatter_start(j, b):
        pltpu.async_copy(msgs[b], acc.at[didx2.at[j]], ssem[b], add=True)

    def scatter_wait(j, b):
        pltpu.make_async_copy(msgs[b], acc.at[didx2.at[j]], ssem[b]).wait()

    for b in range(_LEAD):
        gather_start(b, b)

    def slot(j, b):
        gather_wait(j, b)
        scatter_start(j, b)
        pb = (b - _LEAD) % _NBUF

        @pl.when(j >= _LEAD)
        def _():
            scatter_wait(j - _LEAD, pb)

        @pl.when(j + _LEAD < nblk)
        def _():
            gather_start(j + _LEAD, pb)

    def body(k, carry):
        for b in range(_NBUF):
            slot(k * _NBUF + b, b)
        return carry

    lax.fori_loop(0, nblk // _NBUF, body, 0)
    for j in range(nblk - _LEAD, nblk):
        scatter_wait(j, j % _NBUF)


@functools.lru_cache(maxsize=None)
def _make_sc(n, e):
    nblk = e // _NS // _BLK
    nvec = _RCH * _W // 16

    @functools.partial(
        pl.kernel,
        mesh=_sc_mesh(),
        out_type=jax.ShapeDtypeStruct((n, 2), jnp.float32),
        compiler_params=_sc_params(),
        scratch_types=[
            pltpu.VMEM((nblk, _BLK), jnp.int32),
            pltpu.VMEM((nblk, _BLK), jnp.int32),
            pltpu.VMEM((_BLK, _W), jnp.float32),
            [pltpu.VMEM((_BLK, _W), jnp.float32)] * _NBUF,
            pltpu.VMEM((_RCH, _W), jnp.float32),
            pltpu.VMEM((_RCH, _W), jnp.float32),
            pltpu.VMEM((_RCH, _W), jnp.float32),
            pltpu.VMEM((_RCH, _W), jnp.float32),
            pltpu.VMEM((_RCH, 2), jnp.float32),
            pltpu.VMEM((8, _W), jnp.float32),
            pltpu.VMEM_SHARED((n, _W), jnp.float32),
            pltpu.VMEM_SHARED((n, _W), jnp.float32),
            pltpu.VMEM_SHARED((n, _W), jnp.float32),
            pltpu.VMEM_SHARED((n, _W), jnp.float32),
            [pltpu.SemaphoreType.DMA] * 2,
            [pltpu.SemaphoreType.DMA] * _NBUF,
            [pltpu.SemaphoreType.DMA] * _NBUF,
        ],
    )
    def sc(ei_hbm, zt_hbm, zeros_hbm, ones_hbm, cb_hbm, out_hbm,
           sidxf, didxf, ones_v, msgs, ca, cb, cc, cd, outc, cbv,
           dvtab, gtab, acc1, acc2, dsem, gsem, ssem):
        c = lax.axis_index("c")
        s = lax.axis_index("s")
        pltpu.sync_copy(ei_hbm.at[0, s], sidxf)
        pltpu.sync_copy(ei_hbm.at[1, s], didxf)
        pltpu.sync_copy(ones_hbm, ones_v)
        pltpu.sync_copy(cb_hbm, cbv)
        _chunks_foreach_tile(s, n, lambda cid, r0: pltpu.sync_copy(
            zeros_hbm.at[pl.ds(r0, _RCH)], dvtab.at[pl.ds(r0, _RCH)]))
        plsc.subcore_barrier()

        def dslot(j, b):
            pltpu.async_copy(ones_v, dvtab.at[didxf.at[j]], dsem[b], add=True)

            @pl.when(j >= 2)
            def _():
                pltpu.make_async_copy(
                    ones_v, dvtab.at[didxf.at[j - 2]], dsem[b]).wait()

        def dbody(k, carry):
            dslot(2 * k, 0)
            dslot(2 * k + 1, 1)
            return carry

        lax.fori_loop(0, nblk // 2, dbody, 0)
        for b in range(2):
            pltpu.make_async_copy(
                ones_v, dvtab.at[didxf.at[nblk - 2 + b]], dsem[b]).wait()
        plsc.subcore_barrier()

        def build0(cid, r0):
            pltpu.sync_copy(dvtab.at[pl.ds(r0, _RCH)], ca)
            pltpu.sync_copy(zt_hbm.at[pl.ds(r0, _RCH)], cb)

            def vbody(v, carry):
                row, col = _flat_idx(v)
                d = plsc.load_gather(ca, [row, col]) + 1.0
                y = _rsqrt_sc(d)
                z = plsc.load_gather(cb, [row, col])
                plsc.store_scatter(cc, [row, col], y * z)
                plsc.store_scatter(cd, [row, col], y)
                return carry

            lax.fori_loop(0, nvec, vbody, 0)
            pltpu.sync_copy(cd, dvtab.at[pl.ds(r0, _RCH)])
            pltpu.sync_copy(cc, gtab.at[pl.ds(r0, _RCH)])
            pltpu.sync_copy(cc, acc1.at[pl.ds(r0, _RCH)])

        _chunks_foreach_tile(s, n, build0)
        plsc.subcore_barrier()

        _prop_ring(nblk, gtab, sidxf, didxf, acc1, msgs, gsem, ssem)
        plsc.subcore_barrier()

        def build1(cid, r0):
            pltpu.sync_copy(acc1.at[pl.ds(r0, _RCH)], ca)
            pltpu.sync_copy(dvtab.at[pl.ds(r0, _RCH)], cd)

            def vbody(v, carry):
                row, col = _flat_idx(v)
                t1 = plsc.load_gather(ca, [row, col])
                dv = plsc.load_gather(cd, [row, col])
                plsc.store_scatter(cc, [row, col], dv * dv * t1)
                plsc.store_scatter(cb, [row, col], dv * t1)
                return carry

            lax.fori_loop(0, nvec, vbody, 0)
            pltpu.sync_copy(cc, gtab.at[pl.ds(r0, _RCH)])
            pltpu.sync_copy(cc, acc2.at[pl.ds(r0, _RCH)])
            pltpu.sync_copy(cb, acc1.at[pl.ds(r0, _RCH)])

        _chunks_foreach_tile(s, n, build1)
        plsc.subcore_barrier()

        _prop_ring(nblk, gtab, sidxf, didxf, acc2, msgs, gsem, ssem)
        plsc.subcore_barrier()

        io = lax.iota(jnp.int32, 16)
        z16 = io * 0

        def finish(cid, r0):
            @pl.when(cid % _NC == c)
            def _():
                pltpu.sync_copy(acc2.at[pl.ds(r0, _RCH)], ca)
                pltpu.sync_copy(dvtab.at[pl.ds(r0, _RCH)], cd)
                pltpu.sync_copy(acc1.at[pl.ds(r0, _RCH)], cb)
                c0 = plsc.load_gather(cbv, [z16, z16])
                c1 = plsc.load_gather(cbv, [z16, z16 + 1])
                b20 = plsc.load_gather(cbv, [z16 + 1, z16])
                b21 = plsc.load_gather(cbv, [z16 + 1, z16 + 1])

                def vbody(v, carry):
                    ridx = 16 * v + io
                    t20 = plsc.load_gather(ca, [ridx, z16])
                    t21 = plsc.load_gather(ca, [ridx, z16 + 1])
                    dv = plsc.load_gather(cd, [ridx, z16])
                    tx = plsc.load_gather(cb, [ridx, z16 + 2])
                    y0 = dv * t20 + tx * c0 + b20
                    y1 = dv * t21 + tx * c1 + b21
                    m = jnp.maximum(y0, y1)
                    e0 = jnp.exp(y0 - m)
                    e1 = jnp.exp(y1 - m)
                    tot = e0 + e1
                    plsc.store_scatter(outc, [ridx, z16], e0 / tot)
                    plsc.store_scatter(outc, [ridx, z16 + 1], e1 / tot)
                    return carry

                lax.fori_loop(0, _RCH // 16, vbody, 0)
                pltpu.sync_copy(outc, out_hbm.at[pl.ds(r0, _RCH)])

        _chunks_foreach_tile(s, n, finish)

    return sc


def _tc_stage_a(x, w1, w2p, b1r, b2p):
    n, d = x.shape

    def ka(x_ref, w1_ref, w2p_ref, b1_ref, b2_ref, zt_ref, cb_ref):
        w12 = jnp.dot(w1_ref[...], w2p_ref[...],
                      preferred_element_type=jnp.float32)
        z16 = jnp.dot(x_ref[...], w12, preferred_element_type=jnp.float32)
        col = lax.broadcasted_iota(jnp.int32, z16.shape, 1)
        zt_ref[...] = z16 + jnp.where(col == 2, 1.0, 0.0)
        c16 = jnp.dot(b1_ref[...], w2p_ref[...],
                      preferred_element_type=jnp.float32)
        row = lax.broadcasted_iota(jnp.int32, (8, _W), 0)
        cb_ref[...] = jnp.where(row == 0, c16, jnp.where(row == 1,
                                                         b2_ref[...], 0.0))

    return pl.pallas_call(
        ka,
        grid=(n // _R,),
        in_specs=[
            pl.BlockSpec((_R, d), lambda i: (i, 0)),
            pl.BlockSpec((d, d), lambda i: (0, 0)),
            pl.BlockSpec((d, _W), lambda i: (0, 0)),
            pl.BlockSpec((1, d), lambda i: (0, 0)),
            pl.BlockSpec((1, _W), lambda i: (0, 0)),
        ],
        out_specs=[
            pl.BlockSpec((_R, _W), lambda i: (i, 0)),
            pl.BlockSpec((8, _W), lambda i: (0, 0)),
        ],
        out_shape=[
            jax.ShapeDtypeStruct((n, _W), jnp.float32),
            jax.ShapeDtypeStruct((8, _W), jnp.float32),
        ],
    )(x, w1, w2p, b1r, b2p)


def kernel(X, edge_index, W1, b1, W2, b2):
    n, _ = X.shape
    e = edge_index.shape[1]
    nblk = e // _NS // _BLK
    ei = edge_index.reshape(2, _NS, nblk, _BLK)

    w2p = jnp.pad(W2, ((0, 0), (0, _W - W2.shape[1])))
    b1r = b1.reshape(1, -1)
    b2p = jnp.pad(b2.reshape(1, -1), ((0, 0), (0, _W - b2.shape[0])))
    zeros_t = jnp.zeros((n, _W), jnp.float32)
    ones_t = jnp.ones((_BLK, _W), jnp.float32)

    zt, cb = _tc_stage_a(X, W1, w2p, b1r, b2p)
    return _make_sc(n, e)(ei, zt, zeros_t, ones_t, cb)


# speedup vs baseline: 2.0950x; 1.0970x over previous
import functools

import jax
import jax.numpy as jnp
from jax import lax
from jax.experimental import pallas as pl
from jax.experimental.pallas import tpu as pltpu
from jax.experimental.pallas import tpu_sc as plsc

_W = 8
_BLK = 125
_NBUF = 10
_LEAD = 5
_NC = 2
_NS = 16
_R = 1000
_RCH = 400

_MAGIC = 0x5F3759DF


def _sc_mesh():
    return plsc.VectorSubcoreMesh(core_axis_name="c", subcore_axis_name="s")


def _sc_params():
    return pltpu.CompilerParams(
        use_tc_tiling_on_sc=False, needs_layout_passes=False)


def _chunks_foreach_tile(s, n, fn):
    nch = n // _RCH
    trips = (nch + _NS - 1) // _NS

    def body(i, carry):
        cid = s + i * _NS

        @pl.when(cid < nch)
        def _():
            fn(cid, pl.multiple_of(cid * _RCH, 8))

        return carry

    lax.fori_loop(0, trips, body, 0)


def _flat_idx(v):
    io = lax.iota(jnp.int32, 16)
    return 2 * v + (io >> 3), io & 7


def _rsqrt_sc(d):
    i = plsc.bitcast(d, jnp.int32)
    y = plsc.bitcast(_MAGIC - (i >> 1), jnp.float32)
    for _ in range(3):
        y = y * (1.5 - 0.5 * d * y * y)
    return y


def _prop_ring(nblk, gtab, sidx2, didx2, acc, msgs, gsem, ssem):
    def gather_start(j, b):
        pltpu.async_copy(gtab.at[sidx2.at[j]], msgs[b], gsem[b])

    def gather_wait(j, b):
        pltpu.make_async_copy(gtab.at[sidx2.at[j]], msgs[b], gsem[b]).wait()

    def scatter_start(j, b):
        pltpu.async_copy(msgs[b], acc.at[didx2.at[j]], ssem[b], add=True)

    def scatter_wait(j, b):
        pltpu.make_async_copy(msgs[b], acc.at[didx2.at[j]], ssem[b]).wait()

    for b in range(_LEAD):
        gather_start(b, b)

    def slot(j, b):
        gather_wait(j, b)
        scatter_start(j, b)
        pb = (b - _LEAD) % _NBUF

        @pl.when(j >= _LEAD)
        def _():
            scatter_wait(j - _LEAD, pb)

        @pl.when(j + _LEAD < nblk)
        def _():
            gather_start(j + _LEAD, pb)

    def body(k, carry):
        for b in range(_NBUF):
            slot(k * _NBUF + b, b)
        return carry

    lax.fori_loop(0, nblk // _NBUF, body, 0)
    for j in range(nblk - _LEAD, nblk):
        scatter_wait(j, j % _NBUF)


@functools.lru_cache(maxsize=None)
def _make_sc(n, e):
    nblk = e // _NS // _BLK
    nvec = _RCH * _W // 16

    @functools.partial(
        pl.kernel,
        mesh=_sc_mesh(),
        out_type=jax.ShapeDtypeStruct((n, 2), jnp.float32),
        compiler_params=_sc_params(),
        scratch_types=[
            pltpu.VMEM((nblk, _BLK), jnp.int32),
            pltpu.VMEM((nblk, _BLK), jnp.int32),
            pltpu.VMEM((_BLK, _W), jnp.float32),
            [pltpu.VMEM((_BLK, _W), jnp.float32)] * _NBUF,
            pltpu.VMEM((_RCH, _W), jnp.float32),
            pltpu.VMEM((_RCH, _W), jnp.float32),
            pltpu.VMEM((_RCH, _W), jnp.float32),
            pltpu.VMEM((_RCH, _W), jnp.float32),
            pltpu.VMEM((_RCH, 2), jnp.float32),
            pltpu.VMEM((8, _W), jnp.float32),
            pltpu.VMEM_SHARED((n, _W), jnp.float32),
            pltpu.VMEM_SHARED((n, _W), jnp.float32),
            pltpu.VMEM_SHARED((n, _W), jnp.float32),
            pltpu.VMEM_SHARED((n, _W), jnp.float32),
            [pltpu.SemaphoreType.DMA] * 2,
            [pltpu.SemaphoreType.DMA] * _NBUF,
            [pltpu.SemaphoreType.DMA] * _NBUF,
        ],
    )
    def sc(ei_hbm, zt_hbm, zeros_hbm, ones_hbm, cb_hbm, out_hbm,
           sidxf, didxf, ones_v, msgs, ca, cb, cc, cd, outc, cbv,
           dvtab, gtab, acc1, acc2, dsem, gsem, ssem):
        c = lax.axis_index("c")
        s = lax.axis_index("s")
        pltpu.sync_copy(ei_hbm.at[0, s], sidxf)
        pltpu.sync_copy(ei_hbm.at[1, s], didxf)
        pltpu.sync_copy(ones_hbm, ones_v)
        pltpu.sync_copy(cb_hbm, cbv)
        _chunks_foreach_tile(s, n, lambda cid, r0: pltpu.sync_copy(
            zeros_hbm.at[pl.ds(r0, _RCH)], dvtab.at[pl.ds(r0, _RCH)]))
        plsc.subcore_barrier()

        def dslot(j, b):
            pltpu.async_copy(ones_v, dvtab.at[didxf.at[j]], dsem[b], add=True)

            @pl.when(j >= 2)
            def _():
                pltpu.make_async_copy(
                    ones_v, dvtab.at[didxf.at[j - 2]], dsem[b]).wait()

        def dbody(k, carry):
            dslot(2 * k, 0)
            dslot(2 * k + 1, 1)
            return carry

        lax.fori_loop(0, nblk // 2, dbody, 0)
        for b in range(2):
            pltpu.make_async_copy(
                ones_v, dvtab.at[didxf.at[nblk - 2 + b]], dsem[b]).wait()
        plsc.subcore_barrier()

        def build0(cid, r0):
            pltpu.sync_copy(dvtab.at[pl.ds(r0, _RCH)], ca)
            pltpu.sync_copy(zt_hbm.at[pl.ds(r0, _RCH)], cb)

            def vbody(v, carry):
                row, col = _flat_idx(v)
                d = plsc.load_gather(ca, [row, col]) + 1.0
                y = _rsqrt_sc(d)
                z = plsc.load_gather(cb, [row, col])
                plsc.store_scatter(cc, [row, col], y * z)
                plsc.store_scatter(cd, [row, col], y)
                return carry

            lax.fori_loop(0, nvec, vbody, 0)
            pltpu.sync_copy(cd, dvtab.at[pl.ds(r0, _RCH)])
            pltpu.sync_copy(cc, gtab.at[pl.ds(r0, _RCH)])
            pltpu.sync_copy(cc, acc1.at[pl.ds(r0, _RCH)])

        _chunks_foreach_tile(s, n, build0)
        plsc.subcore_barrier()

        _prop_ring(nblk, gtab, sidxf, didxf, acc1, msgs, gsem, ssem)
        plsc.subcore_barrier()

        def build1(cid, r0):
            pltpu.sync_copy(acc1.at[pl.ds(r0, _RCH)], ca)
            pltpu.sync_copy(dvtab.at[pl.ds(r0, _RCH)], cd)

            def vbody(v, carry):
                row, col = _flat_idx(v)
                t1 = plsc.load_gather(ca, [row, col])
                dv = plsc.load_gather(cd, [row, col])
                plsc.store_scatter(cc, [row, col], dv * dv * t1)
                plsc.store_scatter(cb, [row, col], dv * t1)
                return carry

            lax.fori_loop(0, nvec, vbody, 0)
            pltpu.sync_copy(cc, gtab.at[pl.ds(r0, _RCH)])
            pltpu.sync_copy(cc, acc2.at[pl.ds(r0, _RCH)])
            pltpu.sync_copy(cb, acc1.at[pl.ds(r0, _RCH)])

        _chunks_foreach_tile(s, n, build1)
        plsc.subcore_barrier()

        _prop_ring(nblk, gtab, sidxf, didxf, acc2, msgs, gsem, ssem)
        plsc.subcore_barrier()

        io = lax.iota(jnp.int32, 16)
        z16 = io * 0

        def finish(cid, r0):
            @pl.when(cid % _NC == c)
            def _():
                pltpu.sync_copy(acc2.at[pl.ds(r0, _RCH)], ca)
                pltpu.sync_copy(dvtab.at[pl.ds(r0, _RCH)], cd)
                pltpu.sync_copy(acc1.at[pl.ds(r0, _RCH)], cb)
                c0 = plsc.load_gather(cbv, [z16, z16])
                c1 = plsc.load_gather(cbv, [z16, z16 + 1])
                b20 = plsc.load_gather(cbv, [z16 + 1, z16])
                b21 = plsc.load_gather(cbv, [z16 + 1, z16 + 1])

                def vbody(v, carry):
                    ridx = 16 * v + io
                    t20 = plsc.load_gather(ca, [ridx, z16])
                    t21 = plsc.load_gather(ca, [ridx, z16 + 1])
                    dv = plsc.load_gather(cd, [ridx, z16])
                    tx = plsc.load_gather(cb, [ridx, z16 + 2])
                    y0 = dv * t20 + tx * c0 + b20
                    y1 = dv * t21 + tx * c1 + b21
                    m = jnp.maximum(y0, y1)
                    e0 = jnp.exp(y0 - m)
                    e1 = jnp.exp(y1 - m)
                    tot = e0 + e1
                    plsc.store_scatter(outc, [ridx, z16], e0 / tot)
                    plsc.store_scatter(outc, [ridx, z16 + 1], e1 / tot)
                    return carry

                lax.fori_loop(0, _RCH // 16, vbody, 0)
                pltpu.sync_copy(outc, out_hbm.at[pl.ds(r0, _RCH)])

        _chunks_foreach_tile(s, n, finish)

    return sc


def _tc_stage_a(x, w1, w2p, b1r, b2p):
    n, d = x.shape

    def ka(x_ref, w1_ref, w2p_ref, b1_ref, b2_ref, zt_ref, cb_ref):
        w12 = jnp.dot(w1_ref[...], w2p_ref[...],
                      preferred_element_type=jnp.float32)
        z16 = jnp.dot(x_ref[...], w12, preferred_element_type=jnp.float32)
        col = lax.broadcasted_iota(jnp.int32, z16.shape, 1)
        zt_ref[...] = z16 + jnp.where(col == 2, 1.0, 0.0)
        c16 = jnp.dot(b1_ref[...], w2p_ref[...],
                      preferred_element_type=jnp.float32)
        row = lax.broadcasted_iota(jnp.int32, (8, _W), 0)
        cb_ref[...] = jnp.where(row == 0, c16, jnp.where(row == 1,
                                                         b2_ref[...], 0.0))

    return pl.pallas_call(
        ka,
        grid=(n // _R,),
        in_specs=[
            pl.BlockSpec((_R, d), lambda i: (i, 0)),
            pl.BlockSpec((d, d), lambda i: (0, 0)),
            pl.BlockSpec((d, _W), lambda i: (0, 0)),
            pl.BlockSpec((1, d), lambda i: (0, 0)),
            pl.BlockSpec((1, _W), lambda i: (0, 0)),
        ],
        out_specs=[
            pl.BlockSpec((_R, _W), lambda i: (i, 0)),
            pl.BlockSpec((8, _W), lambda i: (0, 0)),
        ],
        out_shape=[
            jax.ShapeDtypeStruct((n, _W), jnp.float32),
            jax.ShapeDtypeStruct((8, _W), jnp.float32),
        ],
    )(x, w1, w2p, b1r, b2p)


def kernel(X, edge_index, W1, b1, W2, b2):
    n, _ = X.shape
    e = edge_index.shape[1]
    nblk = e // _NS // _BLK
    ei = edge_index.reshape(2, _NS, nblk, _BLK)

    w2p = jnp.pad(W2, ((0, 0), (0, _W - W2.shape[1])))
    b1r = b1.reshape(1, -1)
    b2p = jnp.pad(b2.reshape(1, -1), ((0, 0), (0, _W - b2.shape[0])))
    zeros_t = jnp.zeros((n, _W), jnp.float32)
    ones_t = jnp.ones((_BLK, _W), jnp.float32)

    zt, cb = _tc_stage_a(X, W1, w2p, b1r, b2p)
    return _make_sc(n, e)(ei, zt, zeros_t, ones_t, cb)


# speedup vs baseline: 2.0964x; 1.0007x over previous
import functools

import jax
import jax.numpy as jnp
from jax import lax
from jax.experimental import pallas as pl
from jax.experimental.pallas import tpu as pltpu
from jax.experimental.pallas import tpu_sc as plsc

_W = 8
_BLK = 125
_NBUF = 10
_LEAD = 5
_DDEP = 2
_NC = 2
_NS = 16
_R = 1000
_RCH = 400

_MAGIC = 0x5F3759DF


def _sc_mesh():
    return plsc.VectorSubcoreMesh(core_axis_name="c", subcore_axis_name="s")


def _sc_params():
    return pltpu.CompilerParams(
        use_tc_tiling_on_sc=False, needs_layout_passes=False)


def _chunks_foreach_tile(s, n, fn):
    nch = n // _RCH
    trips = (nch + _NS - 1) // _NS

    def body(i, carry):
        cid = s + i * _NS

        @pl.when(cid < nch)
        def _():
            fn(cid, pl.multiple_of(cid * _RCH, 8))

        return carry

    lax.fori_loop(0, trips, body, 0)


def _flat_idx(v):
    io = lax.iota(jnp.int32, 16)
    return 2 * v + (io >> 3), io & 7


def _rsqrt_sc(d):
    i = plsc.bitcast(d, jnp.int32)
    y = plsc.bitcast(_MAGIC - (i >> 1), jnp.float32)
    for _ in range(3):
        y = y * (1.5 - 0.5 * d * y * y)
    return y


def _prop_ring(nblk, gtab, sidx2, didx2, acc, msgs, gsem, ssem):
    def gather_start(j, b):
        pltpu.async_copy(gtab.at[sidx2.at[j]], msgs[b], gsem[b])

    def gather_wait(j, b):
        pltpu.make_async_copy(gtab.at[sidx2.at[j]], msgs[b], gsem[b]).wait()

    def scatter_start(j, b):
        pltpu.async_copy(msgs[b], acc.at[didx2.at[j]], ssem[b], add=True)

    def scatter_wait(j, b):
        pltpu.make_async_copy(msgs[b], acc.at[didx2.at[j]], ssem[b]).wait()

    for b in range(_LEAD):
        gather_start(b, b)

    def slot(j, b):
        gather_wait(j, b)
        scatter_start(j, b)
        pb = (b - _LEAD) % _NBUF

        @pl.when(j >= _LEAD)
        def _():
            scatter_wait(j - _LEAD, pb)

        @pl.when(j + _LEAD < nblk)
        def _():
            gather_start(j + _LEAD, pb)

    def body(k, carry):
        for b in range(_NBUF):
            slot(k * _NBUF + b, b)
        return carry

    lax.fori_loop(0, nblk // _NBUF, body, 0)
    for j in range(nblk - _LEAD, nblk):
        scatter_wait(j, j % _NBUF)


@functools.lru_cache(maxsize=None)
def _make_sc(n, e):
    nblk = e // _NS // _BLK
    nvec = _RCH * _W // 16

    @functools.partial(
        pl.kernel,
        mesh=_sc_mesh(),
        out_type=jax.ShapeDtypeStruct((n, 2), jnp.float32),
        compiler_params=_sc_params(),
        scratch_types=[
            pltpu.VMEM((nblk, _BLK), jnp.int32),
            pltpu.VMEM((nblk, _BLK), jnp.int32),
            pltpu.VMEM((_BLK, _W), jnp.float32),
            [pltpu.VMEM((_BLK, _W), jnp.float32)] * _NBUF,
            pltpu.VMEM((_RCH, _W), jnp.float32),
            pltpu.VMEM((_RCH, _W), jnp.float32),
            pltpu.VMEM((_RCH, _W), jnp.float32),
            pltpu.VMEM((_RCH, _W), jnp.float32),
            pltpu.VMEM((_RCH, 2), jnp.float32),
            pltpu.VMEM((8, _W), jnp.float32),
            pltpu.VMEM_SHARED((n, _W), jnp.float32),
            pltpu.VMEM_SHARED((n, _W), jnp.float32),
            pltpu.VMEM_SHARED((n, _W), jnp.float32),
            pltpu.VMEM_SHARED((n, _W), jnp.float32),
            [pltpu.SemaphoreType.DMA] * 2,
            [pltpu.SemaphoreType.DMA] * _NBUF,
            [pltpu.SemaphoreType.DMA] * _NBUF,
        ],
    )
    def sc(ei_hbm, zt_hbm, zeros_hbm, ones_hbm, cb_hbm, out_hbm,
           sidxf, didxf, ones_v, msgs, ca, cb, cc, cd, outc, cbv,
           dvtab, gtab, acc1, acc2, dsem, gsem, ssem):
        c = lax.axis_index("c")
        s = lax.axis_index("s")
        pltpu.sync_copy(ei_hbm.at[0, s], sidxf)
        pltpu.sync_copy(ei_hbm.at[1, s], didxf)
        pltpu.sync_copy(ones_hbm, ones_v)
        pltpu.sync_copy(cb_hbm, cbv)
        _chunks_foreach_tile(s, n, lambda cid, r0: pltpu.sync_copy(
            zeros_hbm.at[pl.ds(r0, _RCH)], dvtab.at[pl.ds(r0, _RCH)]))
        plsc.subcore_barrier()

        def dslot(j, b):
            pltpu.async_copy(ones_v, dvtab.at[didxf.at[j]], dsem[b], add=True)

            @pl.when(j >= _DDEP)
            def _():
                pltpu.make_async_copy(
                    ones_v, dvtab.at[didxf.at[j - _DDEP]], dsem[b]).wait()

        def dbody(k, carry):
            for b in range(_DDEP):
                dslot(_DDEP * k + b, b)
            return carry

        lax.fori_loop(0, nblk // _DDEP, dbody, 0)
        for b in range(_DDEP):
            pltpu.make_async_copy(
                ones_v, dvtab.at[didxf.at[nblk - _DDEP + b]], dsem[b]).wait()
        plsc.subcore_barrier()

        def build0(cid, r0):
            pltpu.sync_copy(dvtab.at[pl.ds(r0, _RCH)], ca)
            pltpu.sync_copy(zt_hbm.at[pl.ds(r0, _RCH)], cb)

            def vbody(v, carry):
                row, col = _flat_idx(v)
                d = plsc.load_gather(ca, [row, col]) + 1.0
                y = _rsqrt_sc(d)
                z = plsc.load_gather(cb, [row, col])
                plsc.store_scatter(cc, [row, col], y * z)
                plsc.store_scatter(cd, [row, col], y)
                return carry

            lax.fori_loop(0, nvec, vbody, 0)
            pltpu.sync_copy(cd, dvtab.at[pl.ds(r0, _RCH)])
            pltpu.sync_copy(cc, gtab.at[pl.ds(r0, _RCH)])
            pltpu.sync_copy(cc, acc1.at[pl.ds(r0, _RCH)])

        _chunks_foreach_tile(s, n, build0)
        plsc.subcore_barrier()

        _prop_ring(nblk, gtab, sidxf, didxf, acc1, msgs, gsem, ssem)
        plsc.subcore_barrier()

        def build1(cid, r0):
            pltpu.sync_copy(acc1.at[pl.ds(r0, _RCH)], ca)
            pltpu.sync_copy(dvtab.at[pl.ds(r0, _RCH)], cd)

            def vbody(v, carry):
                row, col = _flat_idx(v)
                t1 = plsc.load_gather(ca, [row, col])
                dv = plsc.load_gather(cd, [row, col])
                plsc.store_scatter(cc, [row, col], dv * dv * t1)
                plsc.store_scatter(cb, [row, col], dv * t1)
                return carry

            lax.fori_loop(0, nvec, vbody, 0)
            pltpu.sync_copy(cc, gtab.at[pl.ds(r0, _RCH)])
            pltpu.sync_copy(cc, acc2.at[pl.ds(r0, _RCH)])
            pltpu.sync_copy(cb, acc1.at[pl.ds(r0, _RCH)])

        _chunks_foreach_tile(s, n, build1)
        plsc.subcore_barrier()

        _prop_ring(nblk, gtab, sidxf, didxf, acc2, msgs, gsem, ssem)
        plsc.subcore_barrier()

        io = lax.iota(jnp.int32, 16)
        z16 = io * 0

        def finish(cid, r0):
            @pl.when(cid % _NC == c)
            def _():
                pltpu.sync_copy(acc2.at[pl.ds(r0, _RCH)], ca)
                pltpu.sync_copy(dvtab.at[pl.ds(r0, _RCH)], cd)
                pltpu.sync_copy(acc1.at[pl.ds(r0, _RCH)], cb)
                c0 = plsc.load_gather(cbv, [z16, z16])
                c1 = plsc.load_gather(cbv, [z16, z16 + 1])
                b20 = plsc.load_gather(cbv, [z16 + 1, z16])
                b21 = plsc.load_gather(cbv, [z16 + 1, z16 + 1])

                def vbody(v, carry):
                    ridx = 16 * v + io
                    t20 = plsc.load_gather(ca, [ridx, z16])
                    t21 = plsc.load_gather(ca, [ridx, z16 + 1])
                    dv = plsc.load_gather(cd, [ridx, z16])
                    tx = plsc.load_gather(cb, [ridx, z16 + 2])
                    y0 = dv * t20 + tx * c0 + b20
                    y1 = dv * t21 + tx * c1 + b21
                    m = jnp.maximum(y0, y1)
                    e0 = jnp.exp(y0 - m)
                    e1 = jnp.exp(y1 - m)
                    tot = e0 + e1
                    plsc.store_scatter(outc, [ridx, z16], e0 / tot)
                    plsc.store_scatter(outc, [ridx, z16 + 1], e1 / tot)
                    return carry

                lax.fori_loop(0, _RCH // 16, vbody, 0)
                pltpu.sync_copy(outc, out_hbm.at[pl.ds(r0, _RCH)])

        _chunks_foreach_tile(s, n, finish)

    return sc


def _tc_stage_a(x, w1, w2p, b1r, b2p):
    n, d = x.shape

    def ka(x_ref, w1_ref, w2p_ref, b1_ref, b2_ref, zt_ref, cb_ref):
        w12 = jnp.dot(w1_ref[...], w2p_ref[...],
                      preferred_element_type=jnp.float32)
        z16 = jnp.dot(x_ref[...], w12, preferred_element_type=jnp.float32)
        col = lax.broadcasted_iota(jnp.int32, z16.shape, 1)
        zt_ref[...] = z16 + jnp.where(col == 2, 1.0, 0.0)
        c16 = jnp.dot(b1_ref[...], w2p_ref[...],
                      preferred_element_type=jnp.float32)
        row = lax.broadcasted_iota(jnp.int32, (8, _W), 0)
        cb_ref[...] = jnp.where(row == 0, c16, jnp.where(row == 1,
                                                         b2_ref[...], 0.0))

    return pl.pallas_call(
        ka,
        grid=(n // _R,),
        in_specs=[
            pl.BlockSpec((_R, d), lambda i: (i, 0)),
            pl.BlockSpec((d, d), lambda i: (0, 0)),
            pl.BlockSpec((d, _W), lambda i: (0, 0)),
            pl.BlockSpec((1, d), lambda i: (0, 0)),
            pl.BlockSpec((1, _W), lambda i: (0, 0)),
        ],
        out_specs=[
            pl.BlockSpec((_R, _W), lambda i: (i, 0)),
            pl.BlockSpec((8, _W), lambda i: (0, 0)),
        ],
        out_shape=[
            jax.ShapeDtypeStruct((n, _W), jnp.float32),
            jax.ShapeDtypeStruct((8, _W), jnp.float32),
        ],
    )(x, w1, w2p, b1r, b2p)


def kernel(X, edge_index, W1, b1, W2, b2):
    n, _ = X.shape
    e = edge_index.shape[1]
    nblk = e // _NS // _BLK
    ei = edge_index.reshape(2, _NS, nblk, _BLK)

    w2p = jnp.pad(W2, ((0, 0), (0, _W - W2.shape[1])))
    b1r = b1.reshape(1, -1)
    b2p = jnp.pad(b2.reshape(1, -1), ((0, 0), (0, _W - b2.shape[0])))
    zeros_t = jnp.zeros((n, _W), jnp.float32)
    ones_t = jnp.ones((_BLK, _W), jnp.float32)

    zt, cb = _tc_stage_a(X, W1, w2p, b1r, b2p)
    return _make_sc(n, e)(ei, zt, zeros_t, ones_t, cb)
